# Initial kernel scaffold; baseline (speedup 1.0000x reference)
import jax
import jax.numpy as jnp
from jax.experimental import pallas as pl


def kernel(feats, points, Wq, bq, Wk, bk, Wv, bv, Wd1, bd1, Wd2, bd2, Wg1, bg1, Wg2, bg2, bn_d_g, bn_d_b, bn_g1_g, bn_g1_b, bn_g2_g, bn_g2_b):
    raise NotImplementedError("write your pallas kernel here")


# speedup vs baseline: 4.0532x; 4.0532x over previous
import jax
import jax.numpy as jnp
from jax.experimental import pallas as pl

_B, _N, _DP, _DF, _K = 4, 8192, 3, 64, 16
_EPS = 1e-5
_TQ = 256
_TT = 512
_CNT = float(_B * _N * _K)
_HI = jax.lax.Precision.HIGHEST


def _gelu(x):
    return 0.5 * x * (1.0 + jax.lax.erf(x * 0.7071067811865476))


def _mm(a, b):
    return jax.lax.dot_general(a, b, (((1,), (0,)), ((), ())),
                               preferred_element_type=jnp.float32,
                               precision=_HI)


def _knn_body(pts_ref, fts_ref, q_ref, wd1t_ref, bd1_ref,
              kf_ref, p1_ref, ds_ref):
    b = pl.program_id(0)
    t = pl.program_id(1)
    P = pts_ref[0]
    F = fts_ref[0]
    Q = q_ref[0]
    rn = jnp.sum(P * P, axis=1)
    qn = jnp.sum(Q * Q, axis=1)
    dot = jax.lax.dot_general(Q, P, (((1,), (1,)), ((), ())),
                              preferred_element_type=jnp.float32)
    d2 = qn[:, None] + rn[None, :] - 2.0 * dot
    iota = jax.lax.broadcasted_iota(jnp.int32, (_TQ, _N), 1)
    wd1t = wd1t_ref[...]
    bd1 = bd1_ref[0]
    bf16 = jnp.bfloat16
    F_hi = F.astype(bf16)
    F_lo = (F - F_hi.astype(jnp.float32)).astype(bf16)
    P_hi = P.astype(bf16)
    P_lo = (P - P_hi.astype(jnp.float32)).astype(bf16)

    def _oh_mm(oh, hi, lo):
        return (jax.lax.dot_general(oh, hi, (((1,), (0,)), ((), ())),
                                    preferred_element_type=jnp.float32)
                + jax.lax.dot_general(oh, lo, (((1,), (0,)), ((), ())),
                                      preferred_element_type=jnp.float32))

    ssum = jnp.zeros((_DP,), jnp.float32)
    ssq = jnp.zeros((_DP,), jnp.float32)
    for r in range(_K):
        m = jnp.min(d2, axis=1)
        sel = jnp.where(d2 <= m[:, None], iota, _N)
        idx = jnp.min(sel, axis=1)
        oh = (iota == idx[:, None]).astype(bf16)
        kf_ref[0, :, r, :] = _oh_mm(oh, F_hi, F_lo)
        kp_r = _oh_mm(oh, P_hi, P_lo)
        p1_r = _mm(Q - kp_r, wd1t) + bd1
        p1_ref[0, :, r, :] = p1_r
        ssum = ssum + jnp.sum(p1_r, axis=0)
        ssq = ssq + jnp.sum(p1_r * p1_r, axis=0)
        d2 = jnp.where(oh > 0, jnp.inf, d2)

    @pl.when(jnp.logical_and(b == 0, t == 0))
    def _init():
        ds_ref[...] = jnp.zeros_like(ds_ref)

    ds_ref[...] = ds_ref[...] + jnp.stack([ssum, ssq])


def _bn(stats_ref, g_ref, b_ref, x):
    mean = stats_ref[0]
    var = stats_ref[1]
    return (x - mean) / jnp.sqrt(var + _EPS) * g_ref[0] + b_ref[0]


def _finalize(raw):
    mean = raw[0] / _CNT
    var = raw[1] / _CNT - mean * mean
    return jnp.stack([mean, var])


def _gamma1(fts_ref, kf_ref, p1_ref, ds_ref, wd2t_ref, bd2_ref, dg_ref, db_ref,
            wqt_ref, bq_ref, wkt_ref, bk_ref):
    f = fts_ref[0]
    kf = kf_ref[0]
    p1 = p1_ref[0]
    act = _gelu(_bn(ds_ref, dg_ref, db_ref, p1)).reshape(_TT * _K, _DP)
    pos = (_mm(act, wd2t_ref[...]) + bd2_ref[0]).reshape(_TT, _K, _DF)
    q = _mm(f, wqt_ref[...]) + bq_ref[0]
    k_ = (_mm(kf.reshape(_TT * _K, _DF), wkt_ref[...]) + bk_ref[0]).reshape(_TT, _K, _DF)
    return q[:, None, :] - k_ + pos, pos, kf


def _accum(out_ref, x):
    first = jnp.logical_and(pl.program_id(0) == 0, pl.program_id(1) == 0)

    @pl.when(first)
    def _init():
        out_ref[...] = jnp.zeros_like(out_ref)

    s = jnp.sum(x.reshape(-1, x.shape[-1]), axis=0)
    ss = jnp.sum((x * x).reshape(-1, x.shape[-1]), axis=0)
    out_ref[...] = out_ref[...] + jnp.stack([s, ss])


def _p2_body(fts_ref, kf_ref, p1_ref, ds_ref, wd2t_ref, bd2_ref, dg_ref, db_ref,
             wqt_ref, bq_ref, wkt_ref, bk_ref, g1s_ref):
    g1, _, _ = _gamma1(fts_ref, kf_ref, p1_ref, ds_ref, wd2t_ref, bd2_ref,
                       dg_ref, db_ref, wqt_ref, bq_ref, wkt_ref, bk_ref)
    _accum(g1s_ref, g1)


def _p3_body(fts_ref, kf_ref, p1_ref, ds_ref, wd2t_ref, bd2_ref, dg_ref, db_ref,
             wqt_ref, bq_ref, wkt_ref, bk_ref, g1s_ref, g1g_ref, g1b_ref,
             wg1t_ref, bg1_ref, g2s_ref):
    g1, _, _ = _gamma1(fts_ref, kf_ref, p1_ref, ds_ref, wd2t_ref, bd2_ref,
                       dg_ref, db_ref, wqt_ref, bq_ref, wkt_ref, bk_ref)
    act = _gelu(_bn(g1s_ref, g1g_ref, g1b_ref, g1)).reshape(_TT * _K, _DF)
    g2 = (_mm(act, wg1t_ref[...]) + bg1_ref[0]).reshape(_TT, _K, _DF)
    _accum(g2s_ref, g2)


def _p4_body(fts_ref, kf_ref, p1_ref, ds_ref, wd2t_ref, bd2_ref, dg_ref, db_ref,
             wqt_ref, bq_ref, wkt_ref, bk_ref, g1s_ref, g1g_ref, g1b_ref,
             wg1t_ref, bg1_ref, g2s_ref, g2g_ref, g2b_ref, wg2t_ref, bg2_ref,
             wvt_ref, bv_ref, out_ref):
    g1, pos, kf = _gamma1(fts_ref, kf_ref, p1_ref, ds_ref, wd2t_ref, bd2_ref,
                          dg_ref, db_ref, wqt_ref, bq_ref, wkt_ref, bk_ref)
    act = _gelu(_bn(g1s_ref, g1g_ref, g1b_ref, g1)).reshape(_TT * _K, _DF)
    g2 = (_mm(act, wg1t_ref[...]) + bg1_ref[0]).reshape(_TT, _K, _DF)
    act2 = _gelu(_bn(g2s_ref, g2g_ref, g2b_ref, g2)).reshape(_TT * _K, _DF)
    gamma = (_mm(act2, wg2t_ref[...]) + bg2_ref[0]).reshape(_TT, _K, _DF)
    mx = jnp.max(gamma, axis=1, keepdims=True)
    e = jnp.exp(gamma - mx)
    rho = e / jnp.sum(e, axis=1, keepdims=True)
    value = (_mm(kf.reshape(_TT * _K, _DF), wvt_ref[...]) + bv_ref[0]).reshape(_TT, _K, _DF)
    value = value + pos
    out_ref[0] = jnp.sum(rho * value, axis=1)


def _cspec(shape):
    return pl.BlockSpec(shape, lambda b, t, _s=len(shape): (0,) * _s)


def kernel(feats, points, Wq, bq, Wk, bk, Wv, bv, Wd1, bd1, Wd2, bd2,
           Wg1, bg1, Wg2, bg2, bn_d_g, bn_d_b, bn_g1_g, bn_g1_b, bn_g2_g, bn_g2_b):
    f32 = jnp.float32
    r1 = lambda v: v.reshape(1, -1).astype(f32)
    wqt, wkt, wvt = Wq.T, Wk.T, Wv.T
    wd1t, wd2t, wg1t, wg2t = Wd1.T, Wd2.T, Wg1.T, Wg2.T

    kf, p1, ds_raw = pl.pallas_call(
        _knn_body,
        grid=(_B, _N // _TQ),
        in_specs=[
            pl.BlockSpec((1, _N, _DP), lambda b, t: (b, 0, 0)),
            pl.BlockSpec((1, _N, _DF), lambda b, t: (b, 0, 0)),
            pl.BlockSpec((1, _TQ, _DP), lambda b, t: (b, t, 0)),
            _cspec((_DP, _DP)),
            _cspec((1, _DP)),
        ],
        out_specs=[
            pl.BlockSpec((1, _TQ, _K, _DF), lambda b, t: (b, t, 0, 0)),
            pl.BlockSpec((1, _TQ, _K, _DP), lambda b, t: (b, t, 0, 0)),
            _cspec((2, _DP)),
        ],
        out_shape=[
            jax.ShapeDtypeStruct((_B, _N, _K, _DF), f32),
            jax.ShapeDtypeStruct((_B, _N, _K, _DP), f32),
            jax.ShapeDtypeStruct((2, _DP), f32),
        ],
    )(points, feats, points, wd1t, r1(bd1))

    grid2 = (_B, _N // _TT)
    base_specs = [
        pl.BlockSpec((1, _TT, _DF), lambda b, t: (b, t, 0)),
        pl.BlockSpec((1, _TT, _K, _DF), lambda b, t: (b, t, 0, 0)),
        pl.BlockSpec((1, _TT, _K, _DP), lambda b, t: (b, t, 0, 0)),
        _cspec((2, _DP)),
        _cspec((_DP, _DF)),
        _cspec((1, _DF)),
        _cspec((1, _DP)),
        _cspec((1, _DP)),
        _cspec((_DF, _DF)),
        _cspec((1, _DF)),
        _cspec((_DF, _DF)),
        _cspec((1, _DF)),
    ]
    ds = jnp.stack([ds_raw[0] / _CNT,
                    ds_raw[1] / _CNT - (ds_raw[0] / _CNT) ** 2])
    base_args = (feats, kf, p1, ds, wd2t, r1(bd2), r1(bn_d_g), r1(bn_d_b),
                 wqt, r1(bq), wkt, r1(bk))

    g1s_raw = pl.pallas_call(
        _p2_body, grid=grid2,
        in_specs=list(base_specs),
        out_specs=_cspec((2, _DF)),
        out_shape=jax.ShapeDtypeStruct((2, _DF), f32),
    )(*base_args)
    g1s = jnp.stack([g1s_raw[0] / _CNT,
                     g1s_raw[1] / _CNT - (g1s_raw[0] / _CNT) ** 2])

    p3_specs = base_specs + [
        _cspec((2, _DF)), _cspec((1, _DF)), _cspec((1, _DF)),
        _cspec((_DF, _DF)), _cspec((1, _DF)),
    ]
    p3_args = base_args + (g1s, r1(bn_g1_g), r1(bn_g1_b), wg1t, r1(bg1))

    g2s_raw = pl.pallas_call(
        _p3_body, grid=grid2,
        in_specs=list(p3_specs),
        out_specs=_cspec((2, _DF)),
        out_shape=jax.ShapeDtypeStruct((2, _DF), f32),
    )(*p3_args)
    g2s = jnp.stack([g2s_raw[0] / _CNT,
                     g2s_raw[1] / _CNT - (g2s_raw[0] / _CNT) ** 2])

    p4_specs = p3_specs + [
        _cspec((2, _DF)), _cspec((1, _DF)), _cspec((1, _DF)),
        _cspec((_DF, _DF)), _cspec((1, _DF)),
        _cspec((_DF, _DF)), _cspec((1, _DF)),
    ]
    p4_args = p3_args + (g2s, r1(bn_g2_g), r1(bn_g2_b), wg2t, r1(bg2),
                         wvt, r1(bv))

    out = pl.pallas_call(
        _p4_body, grid=grid2,
        in_specs=list(p4_specs),
        out_specs=pl.BlockSpec((1, _TT, _DF), lambda b, t: (b, t, 0)),
        out_shape=jax.ShapeDtypeStruct((_B, _N, _DF), f32),
    )(*p4_args)
    return out
